# baseline (speedup 1.0000x reference)
import functools

import numpy as np
import jax
import jax.numpy as jnp
from jax import lax
from jax.experimental import pallas as pl
from jax.experimental.pallas import tpu as pltpu
from jax.experimental.pallas import tpu_sc as plsc

_TEMPERATURE = 0.5
_MARGIN = 0.3
_MAX_SAMPLES = 2048


def _threefry2x32(k1, k2, x0, x1):
    k1 = np.uint32(k1); k2 = np.uint32(k2)
    x0 = x0.astype(np.uint32).copy(); x1 = x1.astype(np.uint32).copy()
    ks = [k1, k2, np.uint32(k1 ^ k2 ^ np.uint32(0x1BD11BDA))]
    rot = [np.uint32([13, 15, 26, 6]), np.uint32([17, 29, 16, 24])]

    def rotl(v, r):
        return (v << np.uint32(r)) | (v >> np.uint32(32 - r))

    def rounds(x0, x1, rs):
        for r in rs:
            x0 = x0 + x1
            x1 = rotl(x1, r)
            x1 = x1 ^ x0
        return x0, x1

    x0 = x0 + ks[0]; x1 = x1 + ks[1]
    x0, x1 = rounds(x0, x1, rot[0]); x0 = x0 + ks[1]; x1 = x1 + ks[2] + np.uint32(1)
    x0, x1 = rounds(x0, x1, rot[1]); x0 = x0 + ks[2]; x1 = x1 + ks[0] + np.uint32(2)
    x0, x1 = rounds(x0, x1, rot[0]); x0 = x0 + ks[0]; x1 = x1 + ks[1] + np.uint32(3)
    x0, x1 = rounds(x0, x1, rot[1]); x0 = x0 + ks[1]; x1 = x1 + ks[2] + np.uint32(4)
    x0, x1 = rounds(x0, x1, rot[0]); x0 = x0 + ks[2]; x1 = x1 + ks[0] + np.uint32(5)
    return x0, x1


def _fry_split(key, num=2):
    b1, b2 = _threefry2x32(key[0], key[1],
                           np.zeros(num, np.uint32),
                           np.arange(num, dtype=np.uint32))
    return np.stack([b1, b2], axis=1)


def _fry_bits32(key, n):
    b1, b2 = _threefry2x32(key[0], key[1],
                           np.zeros(n, np.uint32),
                           np.arange(n, dtype=np.uint32))
    return b1 ^ b2


@functools.cache
def _sample_constants(num_edges: int, n_nodes: int):
    key = np.array([0, 42], np.uint32)
    kp, kn = _fry_split(key, 2)
    exponent = 3
    u32max = float(np.iinfo(np.uint32).max)
    num_rounds = int(np.ceil(exponent * np.log(max(1, num_edges))
                             / np.log(u32max)))
    x = np.arange(num_edges, dtype=np.int32)
    k = kp
    for _ in range(num_rounds):
        k, sub = _fry_split(k, 2)
        order = np.argsort(_fry_bits32(sub, num_edges), kind="stable")
        x = x[order]
    perm = x[:_MAX_SAMPLES]
    k1, k2 = _fry_split(kn, 2)
    hi = _fry_bits32(k1, _MAX_SAMPLES)
    lo = _fry_bits32(k2, _MAX_SAMPLES)
    span = np.uint32(n_nodes)
    mult = np.uint32((np.uint64(2**16 % n_nodes) ** np.uint64(2))
                     % np.uint64(n_nodes))
    neg = (((hi % span) * mult + lo % span) % span).astype(np.int32)
    return perm.astype(np.int32), neg


def _xlgather(v, idx):
    dn = lax.GatherDimensionNumbers(
        offset_dims=(), collapsed_slice_dims=(0,), start_index_map=(0,))
    return lax.gather(v, idx[:, None], dn, slice_sizes=(1,),
                      mode=lax.GatherScatterMode.PROMISE_IN_BOUNDS)


def _lanemax(v, lane):
    for s in (8, 4, 2, 1):
        v = jnp.maximum(v, _xlgather(v, lane ^ s))
    return v


def _lanesum(v, lane):
    for s in (8, 4, 2, 1):
        v = v + _xlgather(v, lane ^ s)
    return v


def _tree16(vs, lane, op):
    f = jnp.maximum if op == "max" else (lambda a, b: a + b)
    d = 8
    while len(vs) > 1:
        nxt = []
        for a, b in zip(vs[0::2], vs[1::2]):
            t = f(a, _xlgather(a, lane ^ d))
            u = f(b, _xlgather(b, lane ^ d))
            nxt.append(jnp.where((lane & d) == 0, t, u))
        vs = nxt
        d //= 2
    return vs[0]


def _sqrt_range(x, kmin, kmax):
    s = jnp.full((16,), jnp.float32(1.5 * 2.0 ** kmax))
    for k in reversed(range(kmin, kmax)):
        s = jnp.where(x < jnp.float32(4.0 ** (k + 1)),
                      jnp.float32(1.5 * 2.0 ** k), s)
    for _ in range(3):
        s = jnp.float32(0.5) * (s + x / s)
    return s


@functools.cache
def _sc_loss_kernel(n_nodes: int, n_classes: int, nsamp: int, ncores: int):
    info = plsc.get_sparse_core_info()
    nc, ns, lanes = info.num_cores, info.num_subcores, info.num_lanes
    nc = ncores
    nw = nc * ns
    epw = nsamp // nw
    nchunk = n_classes // lanes
    inv_t2 = jnp.float32(2.0 * (1.0 / _TEMPERATURE) * 0.5)

    mesh = plsc.VectorSubcoreMesh(core_axis_name="c", subcore_axis_name="s",
                                  num_cores=ncores)

    @functools.partial(
        pl.kernel,
        mesh=mesh,
        out_type=jax.ShapeDtypeStruct((nw, lanes), jnp.float32),
        scratch_types=[
            pltpu.VMEM((epw,), jnp.int32),
            pltpu.VMEM((epw,), jnp.int32),
            pltpu.VMEM((epw,), jnp.int32),
            pltpu.VMEM((epw,), jnp.int32),
            pltpu.VMEM((5, epw, n_classes), jnp.float32),
            pltpu.VMEM((8, epw // 8, 8 * lanes), jnp.float32),
            pltpu.VMEM((2, lanes), jnp.float32),
            pltpu.VMEM((lanes,), jnp.float32),
            pltpu.SemaphoreType.DMA,
            pltpu.SemaphoreType.DMA,
        ],
    )
    def k(student, teacher, src_tab, dst_tab, perm_h, neg_h, out_h,
          perm_v, neg_v, src_v, dst_v, rows, acc, tmp, outv,
          sem_i, sem_r):
        wid = lax.axis_index("s") * nc + lax.axis_index("c")
        base = wid * epw

        pltpu.sync_copy(perm_h.at[pl.ds(base, epw)], perm_v)
        pltpu.sync_copy(neg_h.at[pl.ds(base, epw)], neg_v)
        ci0 = pltpu.async_copy(src_tab.at[perm_v], src_v, sem_i)
        ci1 = pltpu.async_copy(dst_tab.at[perm_v], dst_v, sem_i)
        ci0.wait()
        ci1.wait()

        gs = [
            pltpu.async_copy(student.at[src_v], rows.at[0], sem_r),
            pltpu.async_copy(student.at[dst_v], rows.at[1], sem_r),
            pltpu.async_copy(student.at[neg_v], rows.at[2], sem_r),
            pltpu.async_copy(teacher.at[src_v], rows.at[3], sem_r),
            pltpu.async_copy(teacher.at[dst_v], rows.at[4], sem_r),
        ]

        lane = lax.iota(jnp.int32, lanes)
        zeros = jnp.zeros((lanes,), jnp.float32)

        def per_edge(e, carry):
            a2 = [[zeros] * 8, [zeros] * 8]
            for j in range(nchunk):
                sl = pl.ds(j * lanes, lanes)
                p = a2[j & 1]
                es = jnp.exp(inv_t2 * rows[0, e, sl])
                ed = jnp.exp(inv_t2 * rows[1, e, sl])
                en = jnp.exp(inv_t2 * rows[2, e, sl])
                eu = jnp.exp(inv_t2 * rows[3, e, sl])
                ev = jnp.exp(inv_t2 * rows[4, e, sl])
                p[0] = p[0] + es * ed
                p[1] = p[1] + eu * ev
                p[2] = p[2] + es * en
                p[3] = p[3] + es * es
                p[4] = p[4] + ed * ed
                p[5] = p[5] + en * en
                p[6] = p[6] + eu * eu
                p[7] = p[7] + ev * ev
            er = e >> 3
            ec = (e & 7) * lanes
            for ki in range(8):
                acc[ki, er, pl.ds(ec, lanes)] = a2[0][ki] + a2[1][ki]
            return carry

        for c in gs:
            c.wait()
        lax.fori_loop(0, epw, per_edge, 0, unroll=2)

        def per_block(b, carry):
            align, margin = carry

            def load16(ref, ki):
                return [ref[ki, 2 * b + (i >> 3),
                            pl.ds((i & 7) * lanes, lanes)]
                        for i in range(lanes)]

            s = [_tree16(load16(acc, ki), lane, "add") for ki in range(8)]
            okm = jnp.full((lanes,), True)
            for ki in range(3, 8):
                okm = (okm & (s[ki] >= jnp.float32(3e-4))
                       & (s[ki] <= jnp.float32(2.5e11)))
            badf = jnp.where(okm, jnp.float32(0.0), jnp.float32(1.0))
            ok = _lanemax(badf, lane)[0] < jnp.float32(0.5)

            def fast(_):
                q_ss = _sqrt_range(s[3], -6, 18)
                q_sd = _sqrt_range(s[4], -6, 18)
                q_sn = _sqrt_range(s[5], -6, 18)
                q_ts = _sqrt_range(s[6], -6, 18)
                q_td = _sqrt_range(s[7], -6, 18)
                sp = s[0] / (q_ss * q_sd)
                tp = s[1] / (q_ts * q_td)
                sm = s[2] / (q_ss * q_sn)
                da = sp - tp
                tmp[0, :] = da * da
                tmp[1, :] = jnp.maximum(jnp.float32(_MARGIN) + sm - sp,
                                        jnp.float32(0.0))
                return 0

            def slow(_):
                def redo(i, c):
                    al, mg = c
                    e = b * lanes + i
                    ms = []
                    for kk in range(5):
                        m = rows[kk, e, pl.ds(0, lanes)]
                        for j in range(1, nchunk):
                            m = jnp.maximum(m, rows[kk, e,
                                                    pl.ds(j * lanes, lanes)])
                        ms.append(_lanemax(m, lane))
                    nss = zeros; nsd = zeros; nsn = zeros
                    nts = zeros; ntd = zeros
                    dsp = zeros; dsn = zeros; dtp = zeros
                    for j in range(nchunk):
                        sl = pl.ds(j * lanes, lanes)
                        es = jnp.exp(inv_t2 * (rows[0, e, sl] - ms[0]))
                        ed = jnp.exp(inv_t2 * (rows[1, e, sl] - ms[1]))
                        en = jnp.exp(inv_t2 * (rows[2, e, sl] - ms[2]))
                        eu = jnp.exp(inv_t2 * (rows[3, e, sl] - ms[3]))
                        ev = jnp.exp(inv_t2 * (rows[4, e, sl] - ms[4]))
                        nss = nss + es * es
                        nsd = nsd + ed * ed
                        nsn = nsn + en * en
                        nts = nts + eu * eu
                        ntd = ntd + ev * ev
                        dsp = dsp + es * ed
                        dsn = dsn + es * en
                        dtp = dtp + eu * ev
                    a_ss = _lanesum(nss, lane)
                    sp = (_lanesum(dsp, lane)
                          / _sqrt_range(a_ss * _lanesum(nsd, lane), 0, 8))
                    tp = (_lanesum(dtp, lane)
                          / _sqrt_range(_lanesum(nts, lane)
                                        * _lanesum(ntd, lane), 0, 8))
                    sm = (_lanesum(dsn, lane)
                          / _sqrt_range(a_ss * _lanesum(nsn, lane), 0, 8))
                    da = sp - tp
                    al = al + jnp.where(lane == 0, da * da, jnp.float32(0.0))
                    mg = mg + jnp.where(
                        lane == 0,
                        jnp.maximum(jnp.float32(_MARGIN) + sm - sp,
                                    jnp.float32(0.0)),
                        jnp.float32(0.0))
                    return (al, mg)

                al, mg = lax.fori_loop(0, lanes, redo, (zeros, zeros))
                tmp[0, :] = al
                tmp[1, :] = mg
                return 0

            lax.cond(ok, fast, slow, 0)
            return (align + tmp[0, :], margin + tmp[1, :])

        align, margin = lax.fori_loop(0, epw // lanes, per_block,
                                      (zeros, zeros))
        align = _lanesum(align, lane)
        margin = _lanesum(margin, lane)
        outv[...] = (jnp.where(lane == 0, align, jnp.float32(0.0))
                     + jnp.where(lane == 1, margin, jnp.float32(0.0)))
        pltpu.sync_copy(outv, out_h.at[wid])

    return k


def kernel(student_out, teacher_out, edge_index):
    n_nodes, n_classes = student_out.shape
    num_edges = edge_index.shape[1]
    perm, neg = _sample_constants(num_edges, n_nodes)

    student = student_out.astype(jnp.float32)
    teacher = teacher_out.astype(jnp.float32)
    ei = edge_index.astype(jnp.int32)

    kfn = _sc_loss_kernel(n_nodes, n_classes, _MAX_SAMPLES, 2)
    out = kfn(student, teacher, ei[0], ei[1],
              jnp.asarray(perm), jnp.asarray(neg))
    sums = out.sum(axis=0)
    inv = jnp.float32(1.0 / _MAX_SAMPLES)
    return sums[0] * inv + jnp.float32(0.5) * (sums[1] * inv)


# speedup vs baseline: 1.0669x; 1.0669x over previous
import functools

import numpy as np
import jax
import jax.numpy as jnp
from jax import lax
from jax.experimental import pallas as pl
from jax.experimental.pallas import tpu as pltpu
from jax.experimental.pallas import tpu_sc as plsc

_TEMPERATURE = 0.5
_MARGIN = 0.3
_MAX_SAMPLES = 2048


def _threefry2x32(k1, k2, x0, x1):
    k1 = np.uint32(k1); k2 = np.uint32(k2)
    x0 = x0.astype(np.uint32).copy(); x1 = x1.astype(np.uint32).copy()
    ks = [k1, k2, np.uint32(k1 ^ k2 ^ np.uint32(0x1BD11BDA))]
    rot = [np.uint32([13, 15, 26, 6]), np.uint32([17, 29, 16, 24])]

    def rotl(v, r):
        return (v << np.uint32(r)) | (v >> np.uint32(32 - r))

    def rounds(x0, x1, rs):
        for r in rs:
            x0 = x0 + x1
            x1 = rotl(x1, r)
            x1 = x1 ^ x0
        return x0, x1

    x0 = x0 + ks[0]; x1 = x1 + ks[1]
    x0, x1 = rounds(x0, x1, rot[0]); x0 = x0 + ks[1]; x1 = x1 + ks[2] + np.uint32(1)
    x0, x1 = rounds(x0, x1, rot[1]); x0 = x0 + ks[2]; x1 = x1 + ks[0] + np.uint32(2)
    x0, x1 = rounds(x0, x1, rot[0]); x0 = x0 + ks[0]; x1 = x1 + ks[1] + np.uint32(3)
    x0, x1 = rounds(x0, x1, rot[1]); x0 = x0 + ks[1]; x1 = x1 + ks[2] + np.uint32(4)
    x0, x1 = rounds(x0, x1, rot[0]); x0 = x0 + ks[2]; x1 = x1 + ks[0] + np.uint32(5)
    return x0, x1


def _fry_split(key, num=2):
    b1, b2 = _threefry2x32(key[0], key[1],
                           np.zeros(num, np.uint32),
                           np.arange(num, dtype=np.uint32))
    return np.stack([b1, b2], axis=1)


def _fry_bits32(key, n):
    b1, b2 = _threefry2x32(key[0], key[1],
                           np.zeros(n, np.uint32),
                           np.arange(n, dtype=np.uint32))
    return b1 ^ b2


@functools.cache
def _sample_constants(num_edges: int, n_nodes: int):
    key = np.array([0, 42], np.uint32)
    kp, kn = _fry_split(key, 2)
    exponent = 3
    u32max = float(np.iinfo(np.uint32).max)
    num_rounds = int(np.ceil(exponent * np.log(max(1, num_edges))
                             / np.log(u32max)))
    x = np.arange(num_edges, dtype=np.int32)
    k = kp
    for _ in range(num_rounds):
        k, sub = _fry_split(k, 2)
        order = np.argsort(_fry_bits32(sub, num_edges), kind="stable")
        x = x[order]
    perm = x[:_MAX_SAMPLES]
    k1, k2 = _fry_split(kn, 2)
    hi = _fry_bits32(k1, _MAX_SAMPLES)
    lo = _fry_bits32(k2, _MAX_SAMPLES)
    span = np.uint32(n_nodes)
    mult = np.uint32((np.uint64(2**16 % n_nodes) ** np.uint64(2))
                     % np.uint64(n_nodes))
    neg = (((hi % span) * mult + lo % span) % span).astype(np.int32)
    return perm.astype(np.int32), neg


def _xlgather(v, idx):
    dn = lax.GatherDimensionNumbers(
        offset_dims=(), collapsed_slice_dims=(0,), start_index_map=(0,))
    return lax.gather(v, idx[:, None], dn, slice_sizes=(1,),
                      mode=lax.GatherScatterMode.PROMISE_IN_BOUNDS)


def _lanemax(v, lane):
    for s in (8, 4, 2, 1):
        v = jnp.maximum(v, _xlgather(v, lane ^ s))
    return v


def _lanesum(v, lane):
    for s in (8, 4, 2, 1):
        v = v + _xlgather(v, lane ^ s)
    return v


def _tree16(vs, lane, op):
    f = jnp.maximum if op == "max" else (lambda a, b: a + b)
    d = 8
    while len(vs) > 1:
        nxt = []
        for a, b in zip(vs[0::2], vs[1::2]):
            t = f(a, _xlgather(a, lane ^ d))
            u = f(b, _xlgather(b, lane ^ d))
            nxt.append(jnp.where((lane & d) == 0, t, u))
        vs = nxt
        d //= 2
    return vs[0]


def _sqrt_range(x, kmin, kmax):
    s = jnp.full((16,), jnp.float32(1.5 * 2.0 ** kmax))
    for k in reversed(range(kmin, kmax)):
        s = jnp.where(x < jnp.float32(4.0 ** (k + 1)),
                      jnp.float32(1.5 * 2.0 ** k), s)
    for _ in range(3):
        s = jnp.float32(0.5) * (s + x / s)
    return s


@functools.cache
def _sc_loss_kernel(n_nodes: int, n_classes: int, nsamp: int, ncores: int):
    info = plsc.get_sparse_core_info()
    nc, ns, lanes = info.num_cores, info.num_subcores, info.num_lanes
    nc = ncores
    nw = nc * ns
    epw = nsamp // nw
    nchunk = n_classes // lanes
    inv_t2 = jnp.float32(2.0 * (1.0 / _TEMPERATURE) * 0.5)

    mesh = plsc.VectorSubcoreMesh(core_axis_name="c", subcore_axis_name="s",
                                  num_cores=ncores)

    @functools.partial(
        pl.kernel,
        mesh=mesh,
        out_type=jax.ShapeDtypeStruct((nw, lanes), jnp.float32),
        scratch_types=[
            pltpu.VMEM((epw,), jnp.int32),
            pltpu.VMEM((epw,), jnp.int32),
            pltpu.VMEM((epw,), jnp.int32),
            pltpu.VMEM((epw,), jnp.int32),
            pltpu.VMEM((5, epw, n_classes), jnp.float32),
            pltpu.VMEM((8, epw // 8, 8 * lanes), jnp.float32),
            pltpu.VMEM((2, lanes), jnp.float32),
            pltpu.VMEM((lanes,), jnp.float32),
            pltpu.SemaphoreType.DMA,
            pltpu.SemaphoreType.DMA,
        ],
    )
    def k(student, teacher, src_tab, dst_tab, perm_h, neg_h, out_h,
          perm_v, neg_v, src_v, dst_v, rows, acc, tmp, outv,
          sem_i, sem_r):
        wid = lax.axis_index("s") * nc + lax.axis_index("c")
        base = wid * epw

        pltpu.sync_copy(perm_h.at[pl.ds(base, epw)], perm_v)
        pltpu.sync_copy(neg_h.at[pl.ds(base, epw)], neg_v)
        ci0 = pltpu.async_copy(src_tab.at[perm_v], src_v, sem_i)
        ci1 = pltpu.async_copy(dst_tab.at[perm_v], dst_v, sem_i)
        ci0.wait()
        ci1.wait()

        gs = [
            pltpu.async_copy(student.at[src_v], rows.at[0], sem_r),
            pltpu.async_copy(student.at[dst_v], rows.at[1], sem_r),
            pltpu.async_copy(student.at[neg_v], rows.at[2], sem_r),
            pltpu.async_copy(teacher.at[src_v], rows.at[3], sem_r),
            pltpu.async_copy(teacher.at[dst_v], rows.at[4], sem_r),
        ]

        lane = lax.iota(jnp.int32, lanes)
        zeros = jnp.zeros((lanes,), jnp.float32)

        def per_edge(e, carry):
            n_ss = zeros; n_sd = zeros; n_sn = zeros
            n_ts = zeros; n_td = zeros
            d_sp = zeros; d_sn = zeros; d_tp = zeros
            for j in range(nchunk):
                sl = pl.ds(j * lanes, lanes)
                es = jnp.exp(inv_t2 * rows[0, e, sl])
                ed = jnp.exp(inv_t2 * rows[1, e, sl])
                en = jnp.exp(inv_t2 * rows[2, e, sl])
                eu = jnp.exp(inv_t2 * rows[3, e, sl])
                ev = jnp.exp(inv_t2 * rows[4, e, sl])
                n_ss = n_ss + es * es
                n_sd = n_sd + ed * ed
                n_sn = n_sn + en * en
                n_ts = n_ts + eu * eu
                n_td = n_td + ev * ev
                d_sp = d_sp + es * ed
                d_sn = d_sn + es * en
                d_tp = d_tp + eu * ev
            er = e >> 3
            ec = (e & 7) * lanes
            for ki, v in enumerate((d_sp, d_tp, d_sn, n_ss, n_sd,
                                    n_sn, n_ts, n_td)):
                acc[ki, er, pl.ds(ec, lanes)] = v
            return carry

        for c in gs:
            c.wait()
        lax.fori_loop(0, epw, per_edge, 0, unroll=2)

        def per_block(b, carry):
            align, margin = carry

            def load16(ref, ki):
                return [ref[ki, 2 * b + (i >> 3),
                            pl.ds((i & 7) * lanes, lanes)]
                        for i in range(lanes)]

            s = [_tree16(load16(acc, ki), lane, "add") for ki in range(8)]
            okm = jnp.full((lanes,), True)
            for ki in range(3, 8):
                okm = (okm & (s[ki] >= jnp.float32(3e-4))
                       & (s[ki] <= jnp.float32(2.5e11)))
            badf = jnp.where(okm, jnp.float32(0.0), jnp.float32(1.0))
            ok = _lanemax(badf, lane)[0] < jnp.float32(0.5)

            def fast(_):
                q_ss = _sqrt_range(s[3], -6, 18)
                q_sd = _sqrt_range(s[4], -6, 18)
                q_sn = _sqrt_range(s[5], -6, 18)
                q_ts = _sqrt_range(s[6], -6, 18)
                q_td = _sqrt_range(s[7], -6, 18)
                sp = s[0] / (q_ss * q_sd)
                tp = s[1] / (q_ts * q_td)
                sm = s[2] / (q_ss * q_sn)
                da = sp - tp
                tmp[0, :] = da * da
                tmp[1, :] = jnp.maximum(jnp.float32(_MARGIN) + sm - sp,
                                        jnp.float32(0.0))
                return 0

            def slow(_):
                def redo(i, c):
                    al, mg = c
                    e = b * lanes + i
                    ms = []
                    for kk in range(5):
                        m = rows[kk, e, pl.ds(0, lanes)]
                        for j in range(1, nchunk):
                            m = jnp.maximum(m, rows[kk, e,
                                                    pl.ds(j * lanes, lanes)])
                        ms.append(_lanemax(m, lane))
                    nss = zeros; nsd = zeros; nsn = zeros
                    nts = zeros; ntd = zeros
                    dsp = zeros; dsn = zeros; dtp = zeros
                    for j in range(nchunk):
                        sl = pl.ds(j * lanes, lanes)
                        es = jnp.exp(inv_t2 * (rows[0, e, sl] - ms[0]))
                        ed = jnp.exp(inv_t2 * (rows[1, e, sl] - ms[1]))
                        en = jnp.exp(inv_t2 * (rows[2, e, sl] - ms[2]))
                        eu = jnp.exp(inv_t2 * (rows[3, e, sl] - ms[3]))
                        ev = jnp.exp(inv_t2 * (rows[4, e, sl] - ms[4]))
                        nss = nss + es * es
                        nsd = nsd + ed * ed
                        nsn = nsn + en * en
                        nts = nts + eu * eu
                        ntd = ntd + ev * ev
                        dsp = dsp + es * ed
                        dsn = dsn + es * en
                        dtp = dtp + eu * ev
                    a_ss = _lanesum(nss, lane)
                    sp = (_lanesum(dsp, lane)
                          / _sqrt_range(a_ss * _lanesum(nsd, lane), 0, 8))
                    tp = (_lanesum(dtp, lane)
                          / _sqrt_range(_lanesum(nts, lane)
                                        * _lanesum(ntd, lane), 0, 8))
                    sm = (_lanesum(dsn, lane)
                          / _sqrt_range(a_ss * _lanesum(nsn, lane), 0, 8))
                    da = sp - tp
                    al = al + jnp.where(lane == 0, da * da, jnp.float32(0.0))
                    mg = mg + jnp.where(
                        lane == 0,
                        jnp.maximum(jnp.float32(_MARGIN) + sm - sp,
                                    jnp.float32(0.0)),
                        jnp.float32(0.0))
                    return (al, mg)

                al, mg = lax.fori_loop(0, lanes, redo, (zeros, zeros))
                tmp[0, :] = al
                tmp[1, :] = mg
                return 0

            lax.cond(ok, fast, slow, 0)
            return (align + tmp[0, :], margin + tmp[1, :])

        align, margin = lax.fori_loop(0, epw // lanes, per_block,
                                      (zeros, zeros))
        align = _lanesum(align, lane)
        margin = _lanesum(margin, lane)
        outv[...] = (jnp.where(lane == 0, align, jnp.float32(0.0))
                     + jnp.where(lane == 1, margin, jnp.float32(0.0)))
        pltpu.sync_copy(outv, out_h.at[wid])

    return k


def kernel(student_out, teacher_out, edge_index):
    n_nodes, n_classes = student_out.shape
    num_edges = edge_index.shape[1]
    perm, neg = _sample_constants(num_edges, n_nodes)

    student = student_out.astype(jnp.float32)
    teacher = teacher_out.astype(jnp.float32)
    ei = edge_index.astype(jnp.int32)

    kfn = _sc_loss_kernel(n_nodes, n_classes, _MAX_SAMPLES, 2)
    out = kfn(student, teacher, ei[0], ei[1],
              jnp.asarray(perm), jnp.asarray(neg))
    sums = out.sum(axis=0)
    inv = jnp.float32(1.0 / _MAX_SAMPLES)
    return sums[0] * inv + jnp.float32(0.5) * (sums[1] * inv)
